# Initial kernel scaffold; baseline (speedup 1.0000x reference)
import jax
import jax.numpy as jnp
from jax.experimental import pallas as pl


def kernel(x):
    raise NotImplementedError("write your pallas kernel here")


# speedup vs baseline: 3.4785x; 3.4785x over previous
import functools

import jax
import jax.numpy as jnp
from jax import lax
from jax.experimental import pallas as pl
from jax.experimental.pallas import tpu as pltpu
from jax.experimental.pallas import tpu_sc as plsc

B = 16
N = 8192
G = 256
K = 32
L = 16
NCHUNK = N // L
GPT = G // 2
CAP = 112
BUF = 160

_INF = float("inf")
_BIG = 1 << 30


def _sortkv(k, v):
  return plsc.sort_key_val(k, v)


def _merge32(ak, ai, bk, bi):
  rbk = lax.rev(bk, (0,))
  rbi = lax.rev(bi, (0,))
  m = ak <= rbk
  lok = jnp.where(m, ak, rbk)
  loi = jnp.where(m, ai, rbi)
  hik = jnp.where(m, rbk, ak)
  hii = jnp.where(m, rbi, ai)
  lok, loi = _sortkv(lok, loi)
  hik, hii = _sortkv(hik, hii)
  return lok, loi, hik, hii


def _topk_insert(rk0, ri0, rk1, ri1, ck, ci):
  ck, ci = _sortkv(ck, ci)
  rck = lax.rev(ck, (0,))
  rci = lax.rev(ci, (0,))
  m = rk1 <= rck
  lk = jnp.where(m, rk1, rck)
  li = jnp.where(m, ri1, rci)
  lk, li = _sortkv(lk, li)
  return _merge32(rk0, ri0, lk, li)


def _sc_body(xt_hbm, nn_hbm, p_hbm, c_hbm, x0, x1, x2, dists, cbuf, nnst,
             pst, bufk, bufi):
  cid = lax.axis_index("c")
  sid = lax.axis_index("s")
  batch = cid * 8 + lax.rem(sid, jnp.int32(8))
  half = sid // 8
  iota = lax.iota(jnp.int32, L)

  xbase = batch * (3 * N)
  pltpu.sync_copy(xt_hbm.at[pl.ds(xbase, N)], x0)
  pltpu.sync_copy(xt_hbm.at[pl.ds(xbase + N, N)], x1)
  pltpu.sync_copy(xt_hbm.at[pl.ds(xbase + 2 * N, N)], x2)

  def init_chunk(j, carry):
    dists[pl.ds(j * L, L)] = jnp.full((L,), _INF, jnp.float32)
    return carry

  lax.fori_loop(0, NCHUNK, init_chunk, jnp.int32(0))

  def fps_iter(i, far):
    fv = jnp.full((L,), far, jnp.int32)
    cx = plsc.load_gather(x0, [fv])
    cy = plsc.load_gather(x1, [fv])
    cz = plsc.load_gather(x2, [fv])
    cval = jnp.where(iota == 0, cx, jnp.where(iota == 1, cy, cz))
    plsc.store_scatter(cbuf, [i * 3 + iota], cval, mask=iota < 3)

    def chunk(j, carry):
      maxv, maxi = carry
      sl = pl.ds(j * L, L)
      dx = x0[sl] - cx
      dy = x1[sl] - cy
      dz = x2[sl] - cz
      d = (dx * dx + dy * dy) + dz * dz
      dn = jnp.minimum(dists[sl], d)
      dists[sl] = dn
      upd = dn > maxv
      maxv = jnp.where(upd, dn, maxv)
      maxi = jnp.where(upd, j * L + iota, maxi)
      return maxv, maxi

    maxv0 = jnp.full((L,), -_INF, jnp.float32)
    maxi0 = jnp.zeros((L,), jnp.int32)
    maxv, maxi = lax.fori_loop(0, NCHUNK, chunk, (maxv0, maxi0))
    gm = jnp.max(maxv)
    cand = jnp.where(maxv == gm, maxi, _BIG)
    return jnp.min(cand)

  lax.fori_loop(0, G, fps_iter, jnp.int32(0))

  def knn_group(g, carry):
    gg = half * GPT + g
    gv = jnp.full((L,), gg * 3, jnp.int32)
    cx = plsc.load_gather(cbuf, [gv])
    cy = plsc.load_gather(cbuf, [gv + 1])
    cz = plsc.load_gather(cbuf, [gv + 2])

    def dist(j):
      sl = pl.ds(j * L, L)
      dx = x0[sl] - cx
      dy = x1[sl] - cy
      dz = x2[sl] - cz
      return (dx * dx + dy * dy) + dz * dz

    c0k, c0i = _sortkv(dist(0), iota)
    c1k, c1i = _sortkv(dist(1), L + iota)
    rk0, ri0, rk1, ri1 = _merge32(c0k, c0i, c1k, c1i)

    def reselect(rk0, ri0, rk1, ri1, t, pos):
      nq = (pos + (L - 1)) // L

      def fold(q, r):
        k = bufk[pl.ds(q * L, L)]
        i2 = bufi[pl.ds(q * L, L)]
        k = jnp.where(iota < pos - q * L, k, _INF)
        return _topk_insert(*r, k, i2)

      rk0, ri0, rk1, ri1 = lax.fori_loop(0, nq, fold, (rk0, ri0, rk1, ri1))
      return rk0, ri0, rk1, ri1, jnp.max(rk1), jnp.int32(0)

    def chunk(j, carry):
      rk0, ri0, rk1, ri1, t, pos = carry
      d = dist(j)
      m = d < t

      def store(pos):
        cs = plsc.cumsum(m.astype(jnp.int32))
        tgt = pos + cs - 1
        plsc.store_scatter(bufk, [tgt], d, mask=m)
        plsc.store_scatter(bufi, [tgt], j * L + iota, mask=m)
        return pos + jnp.sum(m.astype(jnp.int32))

      pos = lax.cond(jnp.any(m), store, lambda p: p, pos)
      return lax.cond(pos >= CAP, reselect,
                      lambda *a: a, rk0, ri0, rk1, ri1, t, pos)

    carry0 = (rk0, ri0, rk1, ri1, jnp.max(rk1), jnp.int32(0))
    rk0, ri0, rk1, ri1, t, pos = lax.fori_loop(2, NCHUNK, chunk, carry0)
    rk0, ri0, rk1, ri1, t, pos = lax.cond(
        pos > 0, reselect, lambda *a: a, rk0, ri0, rk1, ri1, t, pos)

    nnst[pl.ds(g * K, L)] = ri0
    nnst[pl.ds(g * K + L, L)] = ri1
    iota3 = iota * 3
    for h, ri in ((0, ri0), (1, ri1)):
      off = g * (K * 3) + h * (L * 3)
      plsc.store_scatter(pst, [off + iota3], plsc.load_gather(x0, [ri]))
      plsc.store_scatter(pst, [off + iota3 + 1], plsc.load_gather(x1, [ri]))
      plsc.store_scatter(pst, [off + iota3 + 2], plsc.load_gather(x2, [ri]))
    return carry

  lax.fori_loop(0, GPT, knn_group, jnp.int32(0))

  rowbase = batch * G + half * GPT
  pltpu.sync_copy(nnst, nn_hbm.at[pl.ds(rowbase * K, GPT * K)])
  pltpu.sync_copy(pst, p_hbm.at[pl.ds(rowbase * K * 3, GPT * K * 3)])

  @pl.when(half == 0)
  def _():
    pltpu.sync_copy(cbuf, c_hbm.at[pl.ds(batch * G * 3, G * 3)])


def _sc_call(xt):
  mesh = plsc.VectorSubcoreMesh(core_axis_name="c", subcore_axis_name="s")
  f = pl.kernel(
      _sc_body,
      out_type=(
          jax.ShapeDtypeStruct((B * G * K,), jnp.int32),
          jax.ShapeDtypeStruct((B * G * K * 3,), jnp.float32),
          jax.ShapeDtypeStruct((B * G * 3,), jnp.float32),
      ),
      mesh=mesh,
      compiler_params=pltpu.CompilerParams(needs_layout_passes=False),
      scratch_types=[
          pltpu.VMEM((N,), jnp.float32),
          pltpu.VMEM((N,), jnp.float32),
          pltpu.VMEM((N,), jnp.float32),
          pltpu.VMEM((N,), jnp.float32),
          pltpu.VMEM((G * 3,), jnp.float32),
          pltpu.VMEM((GPT * K,), jnp.int32),
          pltpu.VMEM((GPT * K * 3,), jnp.float32),
          pltpu.VMEM((BUF,), jnp.float32),
          pltpu.VMEM((BUF,), jnp.int32),
      ],
  )
  return f(xt)


@jax.jit
def kernel(x):
  s = x.shape
  xr = x.reshape(-1, s[-2], s[-1])
  xt = xr.transpose(0, 2, 1).reshape(-1)
  nn, p, c = _sc_call(xt)
  p_out = p.reshape(*s[:-2], G, K, 3)
  c_out = c.reshape(*s[:-2], G, 3)
  nn_idx = nn.reshape(B, G, K)
  return (p_out, c_out, nn_idx)
